# baseline (speedup 1.0000x reference)
import functools

import jax
import jax.numpy as jnp
from jax import lax
from jax.experimental import pallas as pl
from jax.experimental.pallas import tpu as pltpu
from jax.experimental.pallas import tpu_sc as plsc

BATCH = 16384
P = 128

_NC = 2
_NS = 16
NW = _NC * _NS
BPW = BATCH // NW
CHUNK = 128
NCH = BPW // CHUNK

_MESH = plsc.VectorSubcoreMesh(core_axis_name="c", subcore_axis_name="s")


@functools.partial(
    pl.kernel,
    mesh=_MESH,
    out_type=jax.ShapeDtypeStruct((BATCH, P), jnp.float32),
    scratch_types=[
        pltpu.VMEM((BPW,), jnp.int32),
        pltpu.VMEM((BPW, P), jnp.float32),
        pltpu.SemaphoreType.DMA,
    ],
)
def _gather_packed(idx_hbm, w_hbm, out_hbm, idx_v, rows_v, sem):
    wid = lax.axis_index("s") * _NC + lax.axis_index("c")
    base = wid * BPW
    pltpu.sync_copy(idx_hbm.at[pl.ds(base, BPW)], idx_v)
    copies = []
    for c in range(NCH):
        sl = pl.ds(c * CHUNK, CHUNK)
        copies.append(
            pltpu.async_copy(w_hbm.at[idx_v.at[sl]], rows_v.at[sl], sem))
    for cp in copies:
        cp.wait()
    pltpu.sync_copy(rows_v, out_hbm.at[pl.ds(base, BPW)])


def _gather_table(ids, w, d):
    n, _ = w.shape
    rp = P // d
    w_packed = w.reshape(n // rp, P)
    packed = _gather_packed(ids // rp, w_packed)
    spans = packed.reshape(BATCH, rp, d)
    sub = (ids % rp).astype(jnp.int32)
    return jnp.take_along_axis(spans, sub[:, None, None], axis=1)[:, 0]


def kernel(instance_ids, light_env_ids, frame_ids, W_inst, W_light, W_app):
    inst = jnp.squeeze(instance_ids).astype(jnp.int32)
    light = jnp.squeeze(light_env_ids).astype(jnp.int32)
    frame = jnp.squeeze(frame_ids).astype(jnp.int32)
    out_i = _gather_table(inst, W_inst, 64)
    out_l = _gather_table(light, W_light, 32)
    out_a = _gather_table(frame, W_app, 32)
    return (out_i, out_l, out_a)


# speedup vs baseline: 1.0386x; 1.0386x over previous
import functools

import jax
import jax.numpy as jnp
from jax import lax
from jax.experimental import pallas as pl
from jax.experimental.pallas import tpu as pltpu
from jax.experimental.pallas import tpu_sc as plsc

BATCH = 16384
P = 128
D_INST = 64

_NC = 2
_NS = 16
NW = _NC * _NS
BPW = BATCH // NW
L = 16
CHUNK = 128
NCH = BPW // CHUNK

_MESH = plsc.VectorSubcoreMesh(core_axis_name="c", subcore_axis_name="s")


@functools.partial(
    pl.kernel,
    mesh=_MESH,
    out_type=jax.ShapeDtypeStruct((BATCH, D_INST), jnp.float32),
    scratch_types=[
        pltpu.VMEM((BPW,), jnp.int32),
        pltpu.SemaphoreType.DMA,
    ],
)
def _gather_rows_inst(idx_hbm, w_hbm, out_hbm, idx_v, sem):
    wid = lax.axis_index("s") * _NC + lax.axis_index("c")
    base = wid * BPW
    pltpu.sync_copy(idx_hbm.at[pl.ds(base, BPW)], idx_v)

    def issue_body(jb, _):
        rvec = idx_v[pl.ds(jb * L, L)]
        for j2 in range(L):
            pltpu.async_copy(
                w_hbm.at[rvec[j2]], out_hbm.at[base + jb * L + j2], sem
            )
        return 0

    lax.fori_loop(0, BPW // L, issue_body, 0)

    def drain_body(jb, _):
        for j2 in range(L):
            pltpu.make_async_copy(
                w_hbm.at[0], out_hbm.at[base + jb * L + j2], sem
            ).wait()
        return 0

    lax.fori_loop(0, BPW // L, drain_body, 0)


@functools.partial(
    pl.kernel,
    mesh=_MESH,
    out_type=jax.ShapeDtypeStruct((BATCH, P), jnp.float32),
    scratch_types=[
        pltpu.VMEM((BPW,), jnp.int32),
        pltpu.VMEM((BPW, P), jnp.float32),
        pltpu.SemaphoreType.DMA,
    ],
)
def _gather_packed(idx_hbm, w_hbm, out_hbm, idx_v, rows_v, sem):
    wid = lax.axis_index("s") * _NC + lax.axis_index("c")
    base = wid * BPW
    pltpu.sync_copy(idx_hbm.at[pl.ds(base, BPW)], idx_v)
    copies = []
    for c in range(NCH):
        sl = pl.ds(c * CHUNK, CHUNK)
        copies.append(
            pltpu.async_copy(w_hbm.at[idx_v.at[sl]], rows_v.at[sl], sem))
    for cp in copies:
        cp.wait()
    pltpu.sync_copy(rows_v, out_hbm.at[pl.ds(base, BPW)])


def _gather_small(ids, w, d):
    n, _ = w.shape
    rp = P // d
    w_packed = w.reshape(n // rp, P)
    packed = _gather_packed(ids // rp, w_packed)
    spans = packed.reshape(BATCH, rp, d)
    sub = (ids % rp).astype(jnp.int32)
    return jnp.take_along_axis(spans, sub[:, None, None], axis=1)[:, 0]


def kernel(instance_ids, light_env_ids, frame_ids, W_inst, W_light, W_app):
    inst = jnp.squeeze(instance_ids).astype(jnp.int32)
    light = jnp.squeeze(light_env_ids).astype(jnp.int32)
    frame = jnp.squeeze(frame_ids).astype(jnp.int32)
    out_i = _gather_rows_inst(inst, W_inst)
    out_l = _gather_small(light, W_light, 32)
    out_a = _gather_small(frame, W_app, 32)
    return (out_i, out_l, out_a)


# speedup vs baseline: 1.1432x; 1.1007x over previous
import functools

import jax
import jax.numpy as jnp
from jax import lax
from jax.experimental import pallas as pl
from jax.experimental.pallas import tpu as pltpu
from jax.experimental.pallas import tpu_sc as plsc

BATCH = 16384

_NC = 2
_NS = 16
NW = _NC * _NS
BPW = BATCH // NW
CHUNK = 128
NCH = BPW // CHUNK

_MESH = plsc.VectorSubcoreMesh(core_axis_name="c", subcore_axis_name="s")


def _make_gather(d):
    @functools.partial(
        pl.kernel,
        mesh=_MESH,
        compiler_params=pltpu.CompilerParams(use_tc_tiling_on_sc=False),
        out_type=jax.ShapeDtypeStruct((BATCH, d), jnp.float32),
        scratch_types=[
            pltpu.VMEM((BPW,), jnp.int32),
            pltpu.VMEM((BPW, d), jnp.float32),
            pltpu.SemaphoreType.DMA,
        ],
    )
    def gather_one(idx_hbm, w_hbm, out_hbm, idx_v, rows_v, sem):
        wid = lax.axis_index("s") * _NC + lax.axis_index("c")
        base = wid * BPW
        pltpu.sync_copy(idx_hbm.at[pl.ds(base, BPW)], idx_v)
        copies = []
        for c in range(NCH):
            sl = pl.ds(c * CHUNK, CHUNK)
            copies.append(
                pltpu.async_copy(w_hbm.at[idx_v.at[sl]], rows_v.at[sl], sem))
        for cp in copies:
            cp.wait()
        pltpu.sync_copy(rows_v, out_hbm.at[pl.ds(base, BPW)])

    return gather_one


_gather_64 = _make_gather(64)
_gather_32 = _make_gather(32)


def kernel(instance_ids, light_env_ids, frame_ids, W_inst, W_light, W_app):
    inst = jnp.squeeze(instance_ids).astype(jnp.int32)
    light = jnp.squeeze(light_env_ids).astype(jnp.int32)
    frame = jnp.squeeze(frame_ids).astype(jnp.int32)
    out_i = _gather_64(inst, W_inst)
    out_l = _gather_32(light, W_light)
    out_a = _gather_32(frame, W_app)
    return (out_i, out_l, out_a)
